# baseline (speedup 1.0000x reference)
import jax
import jax.numpy as jnp
from jax import lax
from jax.experimental import pallas as pl
from jax.experimental.pallas import tpu as pltpu
from jax.experimental.pallas import tpu_sc as plsc

N_FACTORS = 64
BATCH = 16384

NC = 2
NS = 16
L = 16
NW = NC * NS
B_PER_W = BATCH // NW
CHUNK = 128
NCHUNK = B_PER_W // CHUNK
GROUPS = B_PER_W // L


def _sc_body(uids_hbm, mids_hbm, uemb_hbm, memb_hbm, out_hbm,
             uidx_v, midx_v, urows_v, mrows_v, outv, sem):
    wid = lax.axis_index("s") * NC + lax.axis_index("c")
    base = wid * B_PER_W

    for c in range(NCHUNK):
        pltpu.sync_copy(uids_hbm.at[pl.ds(base + c * CHUNK, CHUNK)],
                        uidx_v.at[c])
        pltpu.sync_copy(mids_hbm.at[pl.ds(base + c * CHUNK, CHUNK)],
                        midx_v.at[c])

    copies = []
    for c in range(NCHUNK):
        copies.append(pltpu.async_copy(
            uemb_hbm.at[uidx_v.at[c]],
            urows_v.at[pl.ds(c * CHUNK, CHUNK)], sem))
        copies.append(pltpu.async_copy(
            memb_hbm.at[midx_v.at[c]],
            mrows_v.at[pl.ds(c * CHUNK, CHUNK)], sem))
    for cp in copies:
        cp.wait()

    lane = lax.iota(jnp.int32, L)

    def group(g, _):
        rvec = g * L + lane

        def dstep(d, acc):
            dvec = jnp.full((L,), 0, jnp.int32) + d
            u = plsc.load_gather(urows_v, [rvec, dvec])
            m = plsc.load_gather(mrows_v, [rvec, dvec])
            return acc + u * m

        acc = lax.fori_loop(0, N_FACTORS, dstep,
                            jnp.zeros((L,), jnp.float32))
        outv[pl.ds(pl.multiple_of(g * L, L), L)] = acc
        return 0

    lax.fori_loop(0, GROUPS, group, 0)

    pltpu.sync_copy(outv, out_hbm.at[pl.ds(base, B_PER_W)])


@jax.jit
def _mf_dot(user_ids, movie_ids, user_emb, movie_emb):
    mesh = plsc.VectorSubcoreMesh(core_axis_name="c", subcore_axis_name="s")
    kfn = pl.kernel(
        _sc_body,
        mesh=mesh,
        out_type=jax.ShapeDtypeStruct((BATCH,), jnp.float32),
        compiler_params=pltpu.CompilerParams(
            use_tc_tiling_on_sc=False, needs_layout_passes=False),
        scratch_types=[
            pltpu.VMEM((NCHUNK, CHUNK), jnp.int32),
            pltpu.VMEM((NCHUNK, CHUNK), jnp.int32),
            pltpu.VMEM((B_PER_W, N_FACTORS), jnp.float32),
            pltpu.VMEM((B_PER_W, N_FACTORS), jnp.float32),
            pltpu.VMEM((B_PER_W,), jnp.float32),
            pltpu.SemaphoreType.DMA,
        ],
    )
    return kfn(user_ids, movie_ids, user_emb, movie_emb)


def kernel(user_ids, movie_ids, user_emb, movie_emb):
    return _mf_dot(user_ids.astype(jnp.int32), movie_ids.astype(jnp.int32),
                   user_emb, movie_emb)


# speedup vs baseline: 1.6594x; 1.6594x over previous
import jax
import jax.numpy as jnp
from jax import lax
from jax.experimental import pallas as pl
from jax.experimental.pallas import tpu as pltpu
from jax.experimental.pallas import tpu_sc as plsc

N_FACTORS = 64
BATCH = 16384

NC = 2
NS = 16
L = 16
NW = NC * NS
B_PER_W = BATCH // NW
GROUPS = B_PER_W // L


def _sc_body(uids_hbm, mids_hbm, uemb_hbm, memb_hbm, out_hbm,
             uidx_v, midx_v, urows_v, mrows_v, outv, sems):
    wid = lax.axis_index("s") * NC + lax.axis_index("c")
    base = wid * B_PER_W

    pltpu.sync_copy(uids_hbm.at[pl.ds(base, B_PER_W)], uidx_v)
    pltpu.sync_copy(mids_hbm.at[pl.ds(base, B_PER_W)], midx_v)

    lane = lax.iota(jnp.int32, L)

    def fire(g, buf):
        b0 = pl.multiple_of(g * L, L)
        uvec = uidx_v[pl.ds(b0, L)]
        mvec = midx_v[pl.ds(b0, L)]
        for j in range(L):
            pltpu.async_copy(uemb_hbm.at[pl.ds(uvec[j], 1), :],
                             urows_v.at[buf].at[pl.ds(j, 1), :], sems.at[buf])
            pltpu.async_copy(memb_hbm.at[pl.ds(mvec[j], 1), :],
                             mrows_v.at[buf].at[pl.ds(j, 1), :], sems.at[buf])

    def drain(buf):
        for j in range(L):
            pltpu.make_async_copy(
                uemb_hbm.at[pl.ds(0, 1), :], urows_v.at[buf].at[pl.ds(j, 1), :],
                sems.at[buf]).wait()
            pltpu.make_async_copy(
                memb_hbm.at[pl.ds(0, 1), :], mrows_v.at[buf].at[pl.ds(j, 1), :],
                sems.at[buf]).wait()

    def compute(g, buf):
        def dstep(d, acc):
            dvec = (d + lane) & (N_FACTORS - 1)
            u = plsc.load_gather(urows_v.at[buf], [lane, dvec])
            m = plsc.load_gather(mrows_v.at[buf], [lane, dvec])
            return acc + u * m

        acc = lax.fori_loop(0, N_FACTORS, dstep,
                            jnp.zeros((L,), jnp.float32))
        outv[pl.ds(pl.multiple_of(g * L, L), L)] = acc

    fire(0, 0)

    def group_pair(t, _):
        g0 = t * 2
        drain(0)
        fire(g0 + 1, 1)
        compute(g0, 0)
        drain(1)

        @pl.when(g0 + 2 < GROUPS)
        def _():
            fire(g0 + 2, 0)

        compute(g0 + 1, 1)
        return 0

    lax.fori_loop(0, GROUPS // 2, group_pair, 0)

    pltpu.sync_copy(outv, out_hbm.at[pl.ds(base, B_PER_W)])


@jax.jit
def _mf_dot(user_ids, movie_ids, user_emb, movie_emb):
    mesh = plsc.VectorSubcoreMesh(core_axis_name="c", subcore_axis_name="s")
    kfn = pl.kernel(
        _sc_body,
        mesh=mesh,
        out_type=jax.ShapeDtypeStruct((BATCH,), jnp.float32),
        compiler_params=pltpu.CompilerParams(needs_layout_passes=False),
        scratch_types=[
            pltpu.VMEM((B_PER_W,), jnp.int32),
            pltpu.VMEM((B_PER_W,), jnp.int32),
            pltpu.VMEM((2, L, N_FACTORS), jnp.float32),
            pltpu.VMEM((2, L, N_FACTORS), jnp.float32),
            pltpu.VMEM((B_PER_W,), jnp.float32),
            pltpu.SemaphoreType.DMA((2,)),
        ],
    )
    return kfn(user_ids, movie_ids, user_emb, movie_emb)


def kernel(user_ids, movie_ids, user_emb, movie_emb):
    return _mf_dot(user_ids.astype(jnp.int32), movie_ids.astype(jnp.int32),
                   user_emb, movie_emb)
